# baseline (speedup 1.0000x reference)
import functools

import jax
import jax.numpy as jnp
from jax import lax
from jax.experimental import pallas as pl
from jax.experimental.pallas import tpu as pltpu
from jax.experimental.pallas import tpu_sc as plsc

_L = 16
_BAND = 16
_HALFSC = 256
_TAIL = 1024


def _sc_body(x_hbm, t_hbm, o_hbm,
             di0, di1, do0, do1, td,
             xb0, xb1, tx,
             tin0, tin1, tou0, tou1, ttail,
             inring,
             sdi0, sdi1, sdo0, sdo1,
             sxi0, sxi1, sxo0, sxo1,
             tsi0, tsi1, tso0, tso1):
    batch = x_hbm.shape[0]
    dim = x_hbm.shape[2]
    nvec = dim // _L
    core = lax.axis_index("c")
    sub = lax.axis_index("s")
    wid = sub * 2 + core
    is_tail_tile = wid == 31

    band_d = pl.ds(wid * _BAND, _BAND)
    band_x = pl.ds(512 + core * _HALFSC + sub * _BAND, _BAND)
    myslice = pl.ds(sub * _BAND, _BAND)
    trow = pl.ds(_TAIL, 1)

    dib, dob = (di0, di1), (do0, do1)
    dis, dos = (sdi0, sdi1), (sdo0, sdo1)
    xbufs = (xb0, xb1)
    xis, xos = (sxi0, sxi1), (sxo0, sxo1)
    tin_bufs, tout_bufs = (tin0, tin1), (tou0, tou1)
    tin_sems, tout_sems = (tsi0, tsi1), (tso0, tso1)

    pltpu.sync_copy(t_hbm.at[band_d], td)
    pltpu.sync_copy(t_hbm.at[band_x], tx)

    @pl.when(is_tail_tile)
    def _():
        pltpu.sync_copy(t_hbm.at[trow], ttail)

    def din_copy(b, j):
        return pltpu.make_async_copy(x_hbm.at[b, band_d], dib[j], dis[j])

    def dout_copy(b, j):
        return pltpu.make_async_copy(dob[j], o_hbm.at[b, band_d], dos[j])

    def xin_copy(b, j):
        return pltpu.make_async_copy(
            x_hbm.at[b, band_x], inring.at[j, myslice], xis[j])

    def xout_copy(b, j):
        return pltpu.make_async_copy(
            inring.at[j, myslice], o_hbm.at[b, band_x], xos[j])

    def tin_copy(b, j):
        return pltpu.make_async_copy(x_hbm.at[b, trow], tin_bufs[j], tin_sems[j])

    def tout_copy(b, j):
        return pltpu.make_async_copy(tout_bufs[j], o_hbm.at[b, trow], tout_sems[j])

    for j in range(2):
        din_copy(j, j).start()
        xin_copy(j, j).start()

    @pl.when(is_tail_tile)
    def _():
        tin_copy(0, 0).start()
        tin_copy(1, 1).start()

    def add_rows(dst, src, tref):
        def per_row(r, c2):
            for c in range(nvec):
                sl = pl.ds(c * _L, _L)
                dst[r, sl] = src[r, sl] + tref[r, sl]
            return c2

        lax.fori_loop(0, _BAND, per_row, 0)

    def round_fn(g, carry):
        for j in range(2):
            b = g * 2 + j

            din_copy(b, j).wait()

            @pl.when(b >= 2)
            def _():
                dout_copy(b - 2, j).wait()

            add_rows(dob[j], dib[j], td)
            dout_copy(b, j).start()

            @pl.when(b + 2 < batch)
            def _():
                din_copy(b + 2, j).start()

            xin_copy(b, j).wait()
            pltpu.sync_copy(inring.at[j, myslice], xbufs[j])
            add_rows(xbufs[j], xbufs[j], tx)
            pltpu.sync_copy(xbufs[j], inring.at[j, myslice])
            xout_copy(b, j).start()

            @pl.when(b + 2 < batch)
            def _():
                xout_copy(b, j).wait()
                xin_copy(b + 2, j).start()

            @pl.when(is_tail_tile)
            def _():
                tin_copy(b, j).wait()

                @pl.when(b >= 2)
                def _():
                    tout_copy(b - 2, j).wait()

                for c in range(nvec):
                    sl = pl.ds(c * _L, _L)
                    tout_bufs[j][0, sl] = tin_bufs[j][0, sl] + ttail[0, sl]
                tout_copy(b, j).start()

                @pl.when(b + 2 < batch)
                def _():
                    tin_copy(b + 2, j).start()

        return carry

    lax.fori_loop(0, batch // 2, round_fn, 0)

    for j in range(2):
        dout_copy(batch - 2 + j, j).wait()
        xout_copy(batch - 2 + j, j).wait()


    @pl.when(is_tail_tile)
    def _():
        tout_copy(batch - 2, 0).wait()
        tout_copy(batch - 1, 1).wait()


def kernel(inputs, pos_table):
    batch, positions, dim = inputs.shape
    mesh = plsc.VectorSubcoreMesh(core_axis_name="c", subcore_axis_name="s")
    band = pltpu.VMEM((_BAND, dim), inputs.dtype)
    row = pltpu.VMEM((1, dim), inputs.dtype)
    sem = pltpu.SemaphoreType.DMA
    sc_fn = functools.partial(
        pl.kernel,
        mesh=mesh,
        out_type=jax.ShapeDtypeStruct(inputs.shape, inputs.dtype),
        scratch_types=[
            band, band, band, band, band,
            band, band, band,
            row, row, row, row, row,
            pltpu.VMEM_SHARED((2, _HALFSC, dim), inputs.dtype),
            sem, sem, sem, sem,
            sem, sem, sem, sem,
            sem, sem, sem, sem,
        ],
    )(_sc_body)
    return sc_fn(inputs, pos_table)


# speedup vs baseline: 1.2772x; 1.2772x over previous
import functools

import jax
import jax.numpy as jnp
from jax import lax
from jax.experimental import pallas as pl
from jax.experimental.pallas import tpu as pltpu
from jax.experimental.pallas import tpu_sc as plsc

_L = 16
_BAND = 32
_TAIL = _BAND * 32


def _sc_body(x_hbm, t_hbm, o_hbm,
             in0, in1, ou0, ou1, tbuf,
             tin0, tin1, tou0, tou1, ttail,
             si0, si1, so0, so1, tsi0, tsi1, tso0, tso1):
    batch = x_hbm.shape[0]
    dim = x_hbm.shape[2]
    nvec = dim // _L
    wid = lax.axis_index("s") * 2 + lax.axis_index("c")
    rows = pl.ds(wid * _BAND, _BAND)
    trow = pl.ds(_TAIL, 1)
    is_tail_tile = wid == 31

    in_bufs, out_bufs = (in0, in1), (ou0, ou1)
    in_sems, out_sems = (si0, si1), (so0, so1)
    tin_bufs, tout_bufs = (tin0, tin1), (tou0, tou1)
    tin_sems, tout_sems = (tsi0, tsi1), (tso0, tso1)

    pltpu.sync_copy(t_hbm.at[rows], tbuf)

    @pl.when(is_tail_tile)
    def _():
        pltpu.sync_copy(t_hbm.at[trow], ttail)

    def in_copy(b, j):
        return pltpu.make_async_copy(x_hbm.at[b, rows], in_bufs[j], in_sems[j])

    def out_copy(b, j):
        return pltpu.make_async_copy(out_bufs[j], o_hbm.at[b, rows], out_sems[j])

    def tin_copy(b, j):
        return pltpu.make_async_copy(x_hbm.at[b, trow], tin_bufs[j], tin_sems[j])

    def tout_copy(b, j):
        return pltpu.make_async_copy(tout_bufs[j], o_hbm.at[b, trow], tout_sems[j])

    in_copy(0, 0).start()
    in_copy(1, 1).start()

    @pl.when(is_tail_tile)
    def _():
        tin_copy(0, 0).start()
        tin_copy(1, 1).start()

    def round_fn(g, carry):
        for j in range(2):
            b = g * 2 + j
            in_copy(b, j).wait()

            @pl.when(b >= 2)
            def _():
                out_copy(b - 2, j).wait()

            def per_row(r, c2):
                for c in range(nvec):
                    sl = pl.ds(c * _L, _L)
                    out_bufs[j][r, sl] = in_bufs[j][r, sl] + tbuf[r, sl]
                return c2

            lax.fori_loop(0, _BAND, per_row, 0)
            out_copy(b, j).start()

            @pl.when(b + 2 < batch)
            def _():
                in_copy(b + 2, j).start()

            @pl.when(is_tail_tile)
            def _():
                tin_copy(b, j).wait()

                @pl.when(b >= 2)
                def _():
                    tout_copy(b - 2, j).wait()

                for c in range(nvec):
                    sl = pl.ds(c * _L, _L)
                    tout_bufs[j][0, sl] = tin_bufs[j][0, sl] + ttail[0, sl]
                tout_copy(b, j).start()

                @pl.when(b + 2 < batch)
                def _():
                    tin_copy(b + 2, j).start()

        return carry

    lax.fori_loop(0, batch // 2, round_fn, 0)
    out_copy(batch - 2, 0).wait()
    out_copy(batch - 1, 1).wait()

    @pl.when(is_tail_tile)
    def _():
        tout_copy(batch - 2, 0).wait()
        tout_copy(batch - 1, 1).wait()


def kernel(inputs, pos_table):
    batch, positions, dim = inputs.shape
    mesh = plsc.VectorSubcoreMesh(core_axis_name="c", subcore_axis_name="s")
    band = pltpu.VMEM((_BAND, dim), inputs.dtype)
    row = pltpu.VMEM((1, dim), inputs.dtype)
    sem = pltpu.SemaphoreType.DMA
    sc_fn = functools.partial(
        pl.kernel,
        mesh=mesh,
        out_type=jax.ShapeDtypeStruct(inputs.shape, inputs.dtype),
        scratch_types=[band, band, band, band, band,
                       row, row, row, row, row,
                       sem, sem, sem, sem, sem, sem, sem, sem],
    )(_sc_body)
    return sc_fn(inputs, pos_table)
